# baseline (speedup 1.0000x reference)
import functools

import jax
import jax.numpy as jnp
import numpy as np
from jax import lax
from jax.experimental import pallas as pl
from jax.experimental.pallas import tpu as pltpu
from jax.experimental.pallas import tpu_sc as plsc

HASH_A = 9824516537
HASH_B = 57857966300227
HASH_P = 117130198221199
NUM_W = 1000000
BATCH = 4096
HIST = 50
DIM = 64

NUM_CORES = 2
NUM_SUBCORES = 16
LANES = 16
NUM_WORKERS = NUM_CORES * NUM_SUBCORES
BAGS_PER_WORKER = BATCH // NUM_WORKERS
GROUPS = BAGS_PER_WORKER // LANES
JCHUNK = 25
NCHUNK = HIST // JCHUNK
CHUNK_SLOTS = JCHUNK * DIM * LANES

MASK26 = (1 << 26) - 1
P_HI = HASH_P >> 26
P_LO = HASH_P & MASK26
B_HI = HASH_B >> 26
B_LO = HASH_B & MASK26
C36 = (1 << 36) % NUM_W
C26 = (1 << 26) % NUM_W
_inv = np.float32(1.0 / NUM_W)
for _ in range(3):
    _inv = np.nextafter(_inv, np.float32(0))
INV_WD = float(_inv)

_t0 = np.array([(v * HASH_A) % HASH_P for v in range(1024)], dtype=np.int64)
_t1 = np.array([(v * 1024 * HASH_A) % HASH_P for v in range(1024)],
               dtype=np.int64)
T0_LO = np.asarray(_t0 & MASK26, dtype=np.int32)
T0_HI = np.asarray(_t0 >> 26, dtype=np.int32)
T1_LO = np.asarray(_t1 & MASK26, dtype=np.int32)
T1_HI = np.asarray(_t1 >> 26, dtype=np.int32)


def _add_mod_p(ha, la, hb, lb):
    t = la + lb
    nlo = t & MASK26
    nhi = ha + hb + (t >> 26)
    ut = t - P_LO
    ulo = ut & MASK26
    uhi = ha + (hb - P_HI) + (ut >> 26)
    neg = uhi < 0
    return jnp.where(neg, nhi, uhi), jnp.where(neg, nlo, ulo)


def _step_b(hi, lo):
    t = lo + B_LO
    nlo = t & MASK26
    nhi = hi + (t >> 26) + B_HI
    ut = t - P_LO
    ulo = ut & MASK26
    uhi = hi + (ut >> 26) + (B_HI - P_HI)
    neg = uhi < 0
    return jnp.where(neg, nhi, uhi), jnp.where(neg, nlo, ulo)


def _mod_w2(hi, lo):
    e1 = hi >> 10
    e0 = hi & 1023
    u = e1 * C36 + e0 * C26 + lo
    q = (u.astype(jnp.float32) * jnp.float32(INV_WD)).astype(jnp.int32)
    return u - q * NUM_W


def _sc_body(weight, idx32, t0lo, t0hi, t1lo, t1hi, out,
             bags_idx, ibuf, vbuf, outbuf, v0lo, v0hi, v1lo, v1hi, sem):
    wid = lax.axis_index("s") * NUM_CORES + lax.axis_index("c")
    lane = lax.iota(jnp.int32, LANES)

    pltpu.sync_copy(t0lo, v0lo)
    pltpu.sync_copy(t0hi, v0hi)
    pltpu.sync_copy(t1lo, v1lo)
    pltpu.sync_copy(t1hi, v1hi)

    def compute_chunk(j0, poff):
        def j_body(j, _):
            jfull = jnp.full((LANES,), j, dtype=jnp.int32)
            i_vec = plsc.load_gather(bags_idx, [lane, jfull])
            i0 = i_vec & 1023
            i1 = i_vec >> 10
            shi, slo = _add_mod_p(
                plsc.load_gather(v1hi, [i1]), plsc.load_gather(v1lo, [i1]),
                plsc.load_gather(v0hi, [i0]), plsc.load_gather(v0lo, [i0]))

            def d_body(dg, carry):
                shi, slo = carry
                base = poff + ((j - j0) * DIM + dg * 8) * LANES
                for k in range(8):
                    ibuf[pl.ds(base + k * LANES, LANES)] = _mod_w2(shi, slo)
                    shi, slo = _step_b(shi, slo)
                return shi, slo

            lax.fori_loop(jnp.int32(0), jnp.int32(DIM // 8), d_body, (shi, slo))
            return jnp.int32(0)

        lax.fori_loop(j0, j0 + jnp.int32(JCHUNK), j_body, jnp.int32(0))

    def accumulate_chunk(first, poff):
        def d_body(d, _):
            acc = vbuf[pl.ds(poff + d * LANES, LANES)]
            for j in range(1, JCHUNK):
                acc = acc + vbuf[pl.ds(poff + (j * DIM + d) * LANES, LANES)]
            dfull = jnp.full((LANES,), d, dtype=jnp.int32)
            if first:
                plsc.store_scatter(outbuf, [lane, dfull], acc)
            else:
                plsc.addupdate_scatter(outbuf, [lane, dfull], acc)
            return jnp.int32(0)

        lax.fori_loop(jnp.int32(0), jnp.int32(DIM), d_body, jnp.int32(0))

    def drain_prev(t):
        tp = t - 1
        pprev = tp & 1
        poff = pprev * CHUNK_SLOTS
        pltpu.make_async_copy(
            weight.at[ibuf.at[pl.ds(poff, CHUNK_SLOTS)]],
            vbuf.at[pl.ds(poff, CHUNK_SLOTS)], sem).wait()

        @pl.when((tp & 1) == 0)
        def _():
            accumulate_chunk(True, poff)

        @pl.when((tp & 1) == 1)
        def _():
            accumulate_chunk(False, poff)
            gprev = tp >> 1
            base = wid * BAGS_PER_WORKER + gprev * LANES
            pltpu.sync_copy(outbuf, out.at[pl.ds(base, LANES), :])

    def stage_body(t, _):
        p = t & 1
        poff = p * CHUNK_SLOTS

        @pl.when((t & 1) == 0)
        def _():
            base = wid * BAGS_PER_WORKER + (t >> 1) * LANES
            pltpu.sync_copy(idx32.at[pl.ds(base, LANES), :], bags_idx)

        compute_chunk((t & 1) * JCHUNK, poff)

        @pl.when(t > 0)
        def _():
            drain_prev(t)

        pltpu.async_copy(
            weight.at[ibuf.at[pl.ds(poff, CHUNK_SLOTS)]],
            vbuf.at[pl.ds(poff, CHUNK_SLOTS)], sem)
        return jnp.int32(0)

    nstages = jnp.int32(GROUPS * NCHUNK)
    lax.fori_loop(jnp.int32(0), nstages, stage_body, jnp.int32(0))
    drain_prev(nstages)


@jax.jit
def _hashed_embedding_bag(weight, idx32):
    weight2 = jnp.concatenate([weight, weight])
    mesh = plsc.VectorSubcoreMesh(core_axis_name="c", subcore_axis_name="s")
    f = pl.kernel(
        _sc_body,
        out_type=jax.ShapeDtypeStruct((BATCH, DIM), jnp.float32),
        mesh=mesh,
        compiler_params=pltpu.CompilerParams(needs_layout_passes=False),
        scratch_types=[
            pltpu.VMEM((LANES, HIST), jnp.int32),
            pltpu.VMEM((2 * CHUNK_SLOTS,), jnp.int32),
            pltpu.VMEM((2 * CHUNK_SLOTS,), jnp.float32),
            pltpu.VMEM((LANES, DIM), jnp.float32),
            pltpu.VMEM((1024,), jnp.int32),
            pltpu.VMEM((1024,), jnp.int32),
            pltpu.VMEM((1024,), jnp.int32),
            pltpu.VMEM((1024,), jnp.int32),
            pltpu.SemaphoreType.DMA,
        ],
    )
    return f(weight2, idx32,
             jnp.asarray(T0_LO), jnp.asarray(T0_HI),
             jnp.asarray(T1_LO), jnp.asarray(T1_HI))


def kernel(weight, indices):
    weight = weight.astype(jnp.float32)
    idx32 = indices.astype(jnp.int32)
    return _hashed_embedding_bag(weight, idx32)


# speedup vs baseline: 1.0705x; 1.0705x over previous
import functools

import jax
import jax.numpy as jnp
import numpy as np
from jax import lax
from jax.experimental import pallas as pl
from jax.experimental.pallas import tpu as pltpu
from jax.experimental.pallas import tpu_sc as plsc

HASH_A = 9824516537
HASH_B = 57857966300227
HASH_P = 117130198221199
NUM_W = 1000000
BATCH = 4096
HIST = 50
DIM = 64

NUM_CORES = 2
NUM_SUBCORES = 16
LANES = 16
NUM_WORKERS = NUM_CORES * NUM_SUBCORES
BAGS_PER_WORKER = BATCH // NUM_WORKERS
GROUPS = BAGS_PER_WORKER // LANES
JCHUNK = 25
NCHUNK = HIST // JCHUNK
CHUNK_SLOTS = JCHUNK * DIM * LANES

MASK26 = (1 << 26) - 1
P_HI = HASH_P >> 26
P_LO = HASH_P & MASK26
B_HI = HASH_B >> 26
B_LO = HASH_B & MASK26
C36 = (1 << 36) % NUM_W
C26 = (1 << 26) % NUM_W
_inv = np.float32(1.0 / NUM_W)
for _ in range(3):
    _inv = np.nextafter(_inv, np.float32(0))
INV_WD = float(_inv)

_t0 = np.array([(v * HASH_A) % HASH_P for v in range(1024)], dtype=np.int64)
_t1 = np.array([(v * 1024 * HASH_A) % HASH_P for v in range(1024)],
               dtype=np.int64)
T0_LO = np.asarray(_t0 & MASK26, dtype=np.int32)
T0_HI = np.asarray(_t0 >> 26, dtype=np.int32)
T1_LO = np.asarray(_t1 & MASK26, dtype=np.int32)
T1_HI = np.asarray(_t1 >> 26, dtype=np.int32)


def _add_mod_p(ha, la, hb, lb):
    t = la + lb
    nlo = t & MASK26
    nhi = ha + hb + (t >> 26)
    ut = t - P_LO
    ulo = ut & MASK26
    uhi = ha + (hb - P_HI) + (ut >> 26)
    neg = uhi < 0
    return jnp.where(neg, nhi, uhi), jnp.where(neg, nlo, ulo)


def _step_b(hi, lo):
    t = lo + B_LO
    nlo = t & MASK26
    nhi = hi + (t >> 26) + B_HI
    ut = t - P_LO
    ulo = ut & MASK26
    uhi = hi + (ut >> 26) + (B_HI - P_HI)
    neg = uhi < 0
    return jnp.where(neg, nhi, uhi), jnp.where(neg, nlo, ulo)


def _mod_w2(hi, lo):
    e1 = hi >> 10
    e0 = hi & 1023
    u = e1 * C36 + e0 * C26 + lo
    q = (u.astype(jnp.float32) * jnp.float32(INV_WD)).astype(jnp.int32)
    return u - q * NUM_W


def _sc_body(weight, idx32, t0lo, t0hi, t1lo, t1hi, out,
             bags_idx, ibuf, vbuf, outbuf, v0lo, v0hi, v1lo, v1hi, sem):
    wid = lax.axis_index("s") * NUM_CORES + lax.axis_index("c")
    lane = lax.iota(jnp.int32, LANES)

    pltpu.sync_copy(t0lo, v0lo)
    pltpu.sync_copy(t0hi, v0hi)
    pltpu.sync_copy(t1lo, v1lo)
    pltpu.sync_copy(t1hi, v1hi)

    def compute_chunk(j0, poff):
        def j_body(j, _):
            jfull = jnp.full((LANES,), j, dtype=jnp.int32)
            i_vec = plsc.load_gather(bags_idx, [lane, jfull])
            i0 = i_vec & 1023
            i1 = i_vec >> 10
            shi, slo = _add_mod_p(
                plsc.load_gather(v1hi, [i1]), plsc.load_gather(v1lo, [i1]),
                plsc.load_gather(v0hi, [i0]), plsc.load_gather(v0lo, [i0]))

            def d_body(dg, carry):
                shi, slo = carry
                base = poff + ((j - j0) * DIM + dg * 8) * LANES
                for k in range(8):
                    ibuf[pl.ds(base + k * LANES, LANES)] = _mod_w2(shi, slo)
                    shi, slo = _step_b(shi, slo)
                return shi, slo

            lax.fori_loop(jnp.int32(0), jnp.int32(DIM // 8), d_body, (shi, slo))
            return jnp.int32(0)

        lax.fori_loop(j0, j0 + jnp.int32(JCHUNK), j_body, jnp.int32(0))

    def accumulate_chunk(first, poff):
        def d_body(d, _):
            acc = vbuf[pl.ds(poff + d * LANES, LANES)]
            for j in range(1, JCHUNK):
                acc = acc + vbuf[pl.ds(poff + (j * DIM + d) * LANES, LANES)]
            dfull = jnp.full((LANES,), d, dtype=jnp.int32)
            if first:
                plsc.store_scatter(outbuf, [lane, dfull], acc)
            else:
                plsc.addupdate_scatter(outbuf, [lane, dfull], acc)
            return jnp.int32(0)

        lax.fori_loop(jnp.int32(0), jnp.int32(DIM), d_body, jnp.int32(0))

    def wait_prev(t):
        poff = ((t - 1) & 1) * CHUNK_SLOTS
        pltpu.make_async_copy(
            weight.at[ibuf.at[pl.ds(poff, CHUNK_SLOTS)]],
            vbuf.at[pl.ds(poff, CHUNK_SLOTS)], sem).wait()

    def pool_prev(t):
        tp = t - 1
        poff = (tp & 1) * CHUNK_SLOTS

        @pl.when((tp & 1) == 0)
        def _():
            accumulate_chunk(True, poff)

        @pl.when((tp & 1) == 1)
        def _():
            accumulate_chunk(False, poff)
            gprev = tp >> 1
            base = wid * BAGS_PER_WORKER + gprev * LANES
            pltpu.sync_copy(outbuf, out.at[pl.ds(base, LANES), :])

    def stage_body(t, _):
        poff = (t & 1) * CHUNK_SLOTS

        @pl.when((t & 1) == 0)
        def _():
            base = wid * BAGS_PER_WORKER + (t >> 1) * LANES
            pltpu.sync_copy(idx32.at[pl.ds(base, LANES), :], bags_idx)

        compute_chunk((t & 1) * JCHUNK, poff)

        @pl.when(t > 0)
        def _():
            wait_prev(t)

        pltpu.async_copy(
            weight.at[ibuf.at[pl.ds(poff, CHUNK_SLOTS)]],
            vbuf.at[pl.ds(poff, CHUNK_SLOTS)], sem)

        @pl.when(t > 0)
        def _():
            pool_prev(t)

        return jnp.int32(0)

    nstages = jnp.int32(GROUPS * NCHUNK)
    lax.fori_loop(jnp.int32(0), nstages, stage_body, jnp.int32(0))
    wait_prev(nstages)
    pool_prev(nstages)


@jax.jit
def _hashed_embedding_bag(weight, idx32):
    weight2 = jnp.concatenate([weight, weight])
    mesh = plsc.VectorSubcoreMesh(core_axis_name="c", subcore_axis_name="s")
    f = pl.kernel(
        _sc_body,
        out_type=jax.ShapeDtypeStruct((BATCH, DIM), jnp.float32),
        mesh=mesh,
        compiler_params=pltpu.CompilerParams(needs_layout_passes=False),
        scratch_types=[
            pltpu.VMEM((LANES, HIST), jnp.int32),
            pltpu.VMEM((2 * CHUNK_SLOTS,), jnp.int32),
            pltpu.VMEM((2 * CHUNK_SLOTS,), jnp.float32),
            pltpu.VMEM((LANES, DIM), jnp.float32),
            pltpu.VMEM((1024,), jnp.int32),
            pltpu.VMEM((1024,), jnp.int32),
            pltpu.VMEM((1024,), jnp.int32),
            pltpu.VMEM((1024,), jnp.int32),
            pltpu.SemaphoreType.DMA,
        ],
    )
    return f(weight2, idx32,
             jnp.asarray(T0_LO), jnp.asarray(T0_HI),
             jnp.asarray(T1_LO), jnp.asarray(T1_HI))


def kernel(weight, indices):
    weight = weight.astype(jnp.float32)
    idx32 = indices.astype(jnp.int32)
    return _hashed_embedding_bag(weight, idx32)
